# baseline (speedup 1.0000x reference)
import functools

import jax
import jax.numpy as jnp
from jax import lax
from jax.experimental import pallas as pl
from jax.experimental.pallas import tpu as pltpu
from jax.experimental.pallas import tpu_sc as plsc

B = 512
NCAT = 16
NEXP = 64
EPC = 8
HIDDEN = 256

BBC = 8
BB3 = 256

NW = 32
NU = 21 * (B // 4)
UPW = NU // NW


def _sc_s2d_kernel(obs_hbm, x1_hbm, in_tile, out_tile, sem_g, sem_s):
    i32 = jnp.int32
    wid = lax.axis_index("s") * 2 + lax.axis_index("c")
    s16 = lax.iota(i32, 16)
    patt = 768 * (s16 // 4) + (s16 % 4)
    tail = s16 < 4

    def unit_body(u, carry):
        uid = wid * UPW + u
        p = uid // (B // 4)
        bg = uid - p * (B // 4)
        gh = []
        for r in range(48):
            b2, cc = r // 12, r % 12
            row = (bg * 4 + b2) * 252 + (cc * 21) + p
            gh.append(pltpu.async_copy(
                obs_hbm.at[row], in_tile.at[r], sem_g))
        for h in gh:
            h.wait()

        def row_body(r, c2):
            b2s = r // 12
            ccs = r - b2s * 12
            rv = lax.full((16,), 0, i32) + r
            ob0 = b2s * 192 + ccs * 16
            for i in range(4):
                ob = ob0 + i * 4
                for t in range(6):
                    col = jnp.minimum((i * 84 + 16 * t) + s16, 335)
                    vals = plsc.load_gather(in_tile, [rv, col])
                    oidx = patt + (3072 * t + ob)
                    if t < 5:
                        plsc.store_scatter(out_tile, [oidx], vals)
                    else:
                        plsc.store_scatter(out_tile, [oidx], vals, mask=tail)
            return c2

        lax.fori_loop(0, 48, row_body, 0)

        sh = []
        for q in range(21):
            dst = ((p * 21 + q) * B + bg * 4) * 192
            sh.append(pltpu.async_copy(
                out_tile.at[pl.ds(q * 768, 768)],
                x1_hbm.at[pl.ds(dst, 768)], sem_s))
        for h in sh:
            h.wait()
        return carry

    lax.fori_loop(0, UPW, unit_body, 0)


def _sc_s2d(obs):
    obs_rows = obs.reshape(B * 12 * 21, 336)
    mesh = plsc.VectorSubcoreMesh(core_axis_name="c", subcore_axis_name="s")
    k = functools.partial(
        pl.kernel, _sc_s2d_kernel, mesh=mesh,
        compiler_params=pltpu.CompilerParams(needs_layout_passes=False),
        out_type=jax.ShapeDtypeStruct((21 * 21 * B * 192,), jnp.float32),
        scratch_types=[
            pltpu.VMEM((48, 336), jnp.float32),
            pltpu.VMEM((16128,), jnp.float32),
            pltpu.SemaphoreType.DMA,
            pltpu.SemaphoreType.DMA,
        ],
    )()
    return k(obs_rows).reshape(21, 21, B, 192)


def _convs_kernel(x_ref, w1_ref, b1_ref, w2_ref, b2_ref, w3_ref, b3_ref,
                  f_ref):
    bb = x_ref.shape[2]
    bf16 = jnp.bfloat16
    x = x_ref[...].astype(bf16).reshape(21 * 21 * bb, 192)
    y1 = jnp.dot(x, w1_ref[...], preferred_element_type=jnp.float32)
    y1 = y1.reshape(21, 21, bb, 128)
    o1 = (y1[0:20, 0:20, :, 0:32] + y1[0:20, 1:21, :, 32:64]
          + y1[1:21, 0:20, :, 64:96] + y1[1:21, 1:21, :, 96:128])
    o1 = jnp.maximum(o1 + b1_ref[...].reshape(1, 1, 1, 32), 0.0)
    o1r = o1.astype(bf16).reshape(10, 2, 10, 2, bb, 32)
    x2 = jnp.concatenate(
        [o1r[:, i, :, j] for i in range(2) for j in range(2)], axis=-1)
    y2 = jnp.dot(x2.reshape(100 * bb, 128), w2_ref[...],
                 preferred_element_type=jnp.float32)
    y2 = y2.reshape(10, 10, bb, 256)
    o2 = (y2[0:9, 0:9, :, 0:64] + y2[0:9, 1:10, :, 64:128]
          + y2[1:10, 0:9, :, 128:192] + y2[1:10, 1:10, :, 192:256])
    o2 = jnp.maximum(o2 + b2_ref[...].reshape(1, 1, 1, 64), 0.0)
    y3 = jnp.dot(o2.astype(bf16).reshape(81 * bb, 64), w3_ref[...],
                 preferred_element_type=jnp.float32)
    y3 = y3.reshape(9, 9, bb, 576)
    o3 = 0.0
    for kh in range(3):
        for kw in range(3):
            g = (kh * 3 + kw) * 64
            o3 = o3 + y3[kh:kh + 7, kw:kw + 7, :, g:g + 64]
    o3 = jnp.maximum(o3 + b3_ref[...].reshape(1, 1, 1, 64), 0.0)
    f_ref[...] = o3.astype(bf16).transpose(2, 0, 1, 3).reshape(bb, 3136)


def _dense_kernel(f_ref, pw_ref, pb_ref, cw_ref, cb_ref, w1_ref, b1_ref,
                  w2_ref, b2_ref, oh_ref, hid_ref, log_ref, exp_ref):
    f = f_ref[...]
    hid = jnp.maximum(
        jnp.dot(f, pw_ref[...], preferred_element_type=jnp.float32)
        + pb_ref[...], 0.0)
    hid_ref[...] = hid
    logits = jnp.dot(hid, cw_ref[...], preferred_element_type=jnp.float32) \
        + cb_ref[...]
    log_ref[...] = logits
    m = jnp.max(logits, axis=-1, keepdims=True)
    e = jnp.exp(logits - m)
    probs = e / jnp.sum(e, axis=-1, keepdims=True)

    h1 = jnp.maximum(
        jnp.dot(hid, w1_ref[...], preferred_element_type=jnp.float32)
        + b1_ref[...], 0.0)
    z = jnp.dot(h1, w2_ref[...], preferred_element_type=jnp.float32) \
        + b2_ref[...]
    local = jax.nn.sigmoid(z)
    bb = f.shape[0]
    wts = jnp.broadcast_to(probs[:, :, None], (bb, NCAT, EPC))
    weighted = wts.reshape(bb, NCAT * EPC) * local
    exp_ref[...] = jnp.dot(weighted, oh_ref[...],
                           preferred_element_type=jnp.float32)


def kernel(obs, conv1_w, conv1_b, conv2_w, conv2_b, conv3_w, conv3_b,
           proj_w, proj_b, cat_w, cat_b, head_w1, head_b1, head_w2,
           head_b2, mapping):
    f32 = jnp.float32
    bf16 = jnp.bfloat16
    bsz = obs.shape[0]

    w1a = conv1_w.reshape(32, 12, 2, 4, 2, 4).transpose(1, 3, 5, 2, 4, 0)
    w1a = w1a.reshape(192, 128).astype(bf16)
    b1 = conv1_b.reshape(1, 32)

    w2a = conv2_w.reshape(64, 32, 2, 2, 2, 2).transpose(3, 5, 1, 2, 4, 0)
    w2a = w2a.reshape(128, 256).astype(bf16)
    b2 = conv2_b.reshape(1, 64)

    w3a = conv3_w.transpose(1, 2, 3, 0).reshape(64, 576).astype(bf16)
    b3 = conv3_b.reshape(1, 64)

    pw = proj_w.reshape(64, 7, 7, HIDDEN).transpose(1, 2, 0, 3)
    pw = pw.reshape(7 * 7 * 64, HIDDEN).astype(bf16)
    pb = proj_b.reshape(1, HIDDEN)
    cb = cat_b.reshape(1, NCAT)

    wh1 = head_w1.transpose(1, 0, 2).reshape(HIDDEN, NCAT * (HIDDEN // 2))
    bh1 = head_b1.reshape(1, NCAT * (HIDDEN // 2))
    eye = jnp.eye(NCAT, dtype=f32)
    w2bd = (eye[:, None, :, None] * head_w2[:, :, None, :])
    w2bd = w2bd.reshape(NCAT * (HIDDEN // 2), NCAT * EPC)
    bh2 = head_b2.reshape(1, NCAT * EPC)

    onehot = (mapping.reshape(-1)[:, None]
              == jnp.arange(NEXP, dtype=jnp.int32)[None, :]).astype(f32)

    x1 = _sc_s2d(obs)

    feats = pl.pallas_call(
        _convs_kernel,
        grid=(bsz // BBC,),
        in_specs=[
            pl.BlockSpec((21, 21, BBC, 192), lambda i: (0, 0, i, 0)),
            pl.BlockSpec((192, 128), lambda i: (0, 0)),
            pl.BlockSpec((1, 32), lambda i: (0, 0)),
            pl.BlockSpec((128, 256), lambda i: (0, 0)),
            pl.BlockSpec((1, 64), lambda i: (0, 0)),
            pl.BlockSpec((64, 576), lambda i: (0, 0)),
            pl.BlockSpec((1, 64), lambda i: (0, 0)),
        ],
        out_specs=pl.BlockSpec((BBC, 3136), lambda i: (i, 0)),
        out_shape=jax.ShapeDtypeStruct((bsz, 3136), bf16),
    )(x1, w1a, b1, w2a, b2, w3a, b3)

    hidden, logits, expert = pl.pallas_call(
        _dense_kernel,
        grid=(bsz // BB3,),
        in_specs=[
            pl.BlockSpec((BB3, 3136), lambda i: (i, 0)),
            pl.BlockSpec((3136, HIDDEN), lambda i: (0, 0)),
            pl.BlockSpec((1, HIDDEN), lambda i: (0, 0)),
            pl.BlockSpec((HIDDEN, NCAT), lambda i: (0, 0)),
            pl.BlockSpec((1, NCAT), lambda i: (0, 0)),
            pl.BlockSpec((HIDDEN, 2048), lambda i: (0, 0)),
            pl.BlockSpec((1, 2048), lambda i: (0, 0)),
            pl.BlockSpec((2048, 128), lambda i: (0, 0)),
            pl.BlockSpec((1, 128), lambda i: (0, 0)),
            pl.BlockSpec((128, NEXP), lambda i: (0, 0)),
        ],
        out_specs=[
            pl.BlockSpec((BB3, HIDDEN), lambda i: (i, 0)),
            pl.BlockSpec((BB3, NCAT), lambda i: (i, 0)),
            pl.BlockSpec((BB3, NEXP), lambda i: (i, 0)),
        ],
        out_shape=[
            jax.ShapeDtypeStruct((bsz, HIDDEN), f32),
            jax.ShapeDtypeStruct((bsz, NCAT), f32),
            jax.ShapeDtypeStruct((bsz, NEXP), f32),
        ],
    )(feats, pw, pb, cat_w, cb, wh1, bh1, w2bd, bh2, onehot)

    return (logits, expert, hidden)


# speedup vs baseline: 1.1083x; 1.1083x over previous
import functools

import jax
import jax.numpy as jnp
from jax import lax
from jax.experimental import pallas as pl
from jax.experimental.pallas import tpu as pltpu
from jax.experimental.pallas import tpu_sc as plsc

B = 512
NCAT = 16
NEXP = 64
EPC = 8
HIDDEN = 256

BBC = 8
BB3 = 256

NW = 32
NU = 21 * (B // 4)
UPW = NU // NW


def _sc_s2d_kernel(obs_hbm, x1_hbm, in0, in1, out0, out1, sem_g, sem_s):
    i32 = jnp.int32
    wid = lax.axis_index("s") * 2 + lax.axis_index("c")
    s16 = lax.iota(i32, 16)
    patt = 768 * (s16 // 4) + (s16 % 4)
    tail = s16 < 4

    ins = (in0, in1)
    outs = (out0, out1)

    def fire_gathers(u, slot):
        uid = wid * UPW + u
        p = uid // (B // 4)
        bg = uid - p * (B // 4)
        for r in range(48):
            b2, cc = r // 12, r % 12
            row = (bg * 4 + b2) * 252 + (cc * 21) + p
            pltpu.async_copy(obs_hbm.at[row], ins[slot].at[r], sem_g)

    def drain_gathers(slot):
        pltpu.make_async_copy(
            obs_hbm.at[pl.ds(0, 48)], ins[slot], sem_g).wait()

    def fire_scatters(u, slot):
        uid = wid * UPW + u
        p = uid // (B // 4)
        bg = uid - p * (B // 4)
        for q in range(21):
            dst = ((p * 21 + q) * B + bg * 4) * 192
            pltpu.async_copy(outs[slot].at[pl.ds(q * 768, 768)],
                             x1_hbm.at[pl.ds(dst, 768)], sem_s)

    def drain_scatters(slot):
        pltpu.make_async_copy(outs[slot],
                              x1_hbm.at[pl.ds(0, 16128)], sem_s).wait()

    def permute(slot):
        def row_body(r, c2):
            b2s = r // 12
            ccs = r - b2s * 12
            rv = lax.full((16,), 0, i32) + r
            ob0 = b2s * 192 + ccs * 16
            for i in range(4):
                ob = ob0 + i * 4
                for t in range(6):
                    col = jnp.minimum((i * 84 + 16 * t) + s16, 335)
                    vals = plsc.load_gather(ins[slot], [rv, col])
                    oidx = patt + (3072 * t + ob)
                    if t < 5:
                        plsc.store_scatter(outs[slot], [oidx], vals)
                    else:
                        plsc.store_scatter(outs[slot], [oidx], vals,
                                           mask=tail)
            return c2

        lax.fori_loop(0, 48, row_body, 0)

    fire_gathers(wid * 0, 0)

    def pair_body(g, carry):
        for ph in range(2):
            u = g * 2 + ph
            drain_gathers(ph)

            @pl.when(u + 1 < UPW)
            def _():
                fire_gathers(u + 1, 1 - ph)

            @pl.when(u >= 2)
            def _():
                drain_scatters(ph)

            permute(ph)
            fire_scatters(u, ph)
        return carry

    lax.fori_loop(0, UPW // 2, pair_body, 0)
    drain_scatters(0)
    drain_scatters(1)


def _sc_s2d(obs):
    obs_rows = obs.reshape(B * 12 * 21, 336)
    mesh = plsc.VectorSubcoreMesh(core_axis_name="c", subcore_axis_name="s")
    k = functools.partial(
        pl.kernel, _sc_s2d_kernel, mesh=mesh,
        compiler_params=pltpu.CompilerParams(needs_layout_passes=False),
        out_type=jax.ShapeDtypeStruct((21 * 21 * B * 192,), jnp.float32),
        scratch_types=[
            pltpu.VMEM((48, 336), jnp.float32),
            pltpu.VMEM((48, 336), jnp.float32),
            pltpu.VMEM((16128,), jnp.float32),
            pltpu.VMEM((16128,), jnp.float32),
            pltpu.SemaphoreType.DMA,
            pltpu.SemaphoreType.DMA,
        ],
    )()
    return k(obs_rows).reshape(21, 21, B, 192)


def _convs_kernel(x_ref, w1_ref, b1_ref, w2_ref, b2_ref, w3_ref, b3_ref,
                  f_ref):
    bb = x_ref.shape[2]
    bf16 = jnp.bfloat16
    x = x_ref[...].astype(bf16).reshape(21 * 21 * bb, 192)
    y1 = jnp.dot(x, w1_ref[...], preferred_element_type=jnp.float32)
    y1 = y1.reshape(21, 21, bb, 128)
    o1 = (y1[0:20, 0:20, :, 0:32] + y1[0:20, 1:21, :, 32:64]
          + y1[1:21, 0:20, :, 64:96] + y1[1:21, 1:21, :, 96:128])
    o1 = jnp.maximum(o1 + b1_ref[...].reshape(1, 1, 1, 32), 0.0)
    o1r = o1.astype(bf16).reshape(10, 2, 10, 2, bb, 32)
    x2 = jnp.concatenate(
        [o1r[:, i, :, j] for i in range(2) for j in range(2)], axis=-1)
    y2 = jnp.dot(x2.reshape(100 * bb, 128), w2_ref[...],
                 preferred_element_type=jnp.float32)
    y2 = y2.reshape(10, 10, bb, 256)
    o2 = (y2[0:9, 0:9, :, 0:64] + y2[0:9, 1:10, :, 64:128]
          + y2[1:10, 0:9, :, 128:192] + y2[1:10, 1:10, :, 192:256])
    o2 = jnp.maximum(o2 + b2_ref[...].reshape(1, 1, 1, 64), 0.0)
    y3 = jnp.dot(o2.astype(bf16).reshape(81 * bb, 64), w3_ref[...],
                 preferred_element_type=jnp.float32)
    y3 = y3.reshape(9, 9, bb, 576)
    o3 = 0.0
    for kh in range(3):
        for kw in range(3):
            g = (kh * 3 + kw) * 64
            o3 = o3 + y3[kh:kh + 7, kw:kw + 7, :, g:g + 64]
    o3 = jnp.maximum(o3 + b3_ref[...].reshape(1, 1, 1, 64), 0.0)
    f_ref[...] = o3.astype(bf16).transpose(2, 0, 1, 3).reshape(bb, 3136)


def _dense_kernel(f_ref, pw_ref, pb_ref, cw_ref, cb_ref, w1_ref, b1_ref,
                  w2_ref, b2_ref, oh_ref, hid_ref, log_ref, exp_ref):
    f = f_ref[...]
    hid = jnp.maximum(
        jnp.dot(f, pw_ref[...], preferred_element_type=jnp.float32)
        + pb_ref[...], 0.0)
    hid_ref[...] = hid
    logits = jnp.dot(hid, cw_ref[...], preferred_element_type=jnp.float32) \
        + cb_ref[...]
    log_ref[...] = logits
    m = jnp.max(logits, axis=-1, keepdims=True)
    e = jnp.exp(logits - m)
    probs = e / jnp.sum(e, axis=-1, keepdims=True)

    h1 = jnp.maximum(
        jnp.dot(hid, w1_ref[...], preferred_element_type=jnp.float32)
        + b1_ref[...], 0.0)
    z = jnp.dot(h1, w2_ref[...], preferred_element_type=jnp.float32) \
        + b2_ref[...]
    local = jax.nn.sigmoid(z)
    bb = f.shape[0]
    wts = jnp.broadcast_to(probs[:, :, None], (bb, NCAT, EPC))
    weighted = wts.reshape(bb, NCAT * EPC) * local
    exp_ref[...] = jnp.dot(weighted, oh_ref[...],
                           preferred_element_type=jnp.float32)


def kernel(obs, conv1_w, conv1_b, conv2_w, conv2_b, conv3_w, conv3_b,
           proj_w, proj_b, cat_w, cat_b, head_w1, head_b1, head_w2,
           head_b2, mapping):
    f32 = jnp.float32
    bf16 = jnp.bfloat16
    bsz = obs.shape[0]

    w1a = conv1_w.reshape(32, 12, 2, 4, 2, 4).transpose(1, 3, 5, 2, 4, 0)
    w1a = w1a.reshape(192, 128).astype(bf16)
    b1 = conv1_b.reshape(1, 32)

    w2a = conv2_w.reshape(64, 32, 2, 2, 2, 2).transpose(3, 5, 1, 2, 4, 0)
    w2a = w2a.reshape(128, 256).astype(bf16)
    b2 = conv2_b.reshape(1, 64)

    w3a = conv3_w.transpose(1, 2, 3, 0).reshape(64, 576).astype(bf16)
    b3 = conv3_b.reshape(1, 64)

    pw = proj_w.reshape(64, 7, 7, HIDDEN).transpose(1, 2, 0, 3)
    pw = pw.reshape(7 * 7 * 64, HIDDEN).astype(bf16)
    pb = proj_b.reshape(1, HIDDEN)
    cb = cat_b.reshape(1, NCAT)

    wh1 = head_w1.transpose(1, 0, 2).reshape(HIDDEN, NCAT * (HIDDEN // 2))
    bh1 = head_b1.reshape(1, NCAT * (HIDDEN // 2))
    eye = jnp.eye(NCAT, dtype=f32)
    w2bd = (eye[:, None, :, None] * head_w2[:, :, None, :])
    w2bd = w2bd.reshape(NCAT * (HIDDEN // 2), NCAT * EPC)
    bh2 = head_b2.reshape(1, NCAT * EPC)

    onehot = (mapping.reshape(-1)[:, None]
              == jnp.arange(NEXP, dtype=jnp.int32)[None, :]).astype(f32)

    x1 = _sc_s2d(obs)

    feats = pl.pallas_call(
        _convs_kernel,
        grid=(bsz // BBC,),
        in_specs=[
            pl.BlockSpec((21, 21, BBC, 192), lambda i: (0, 0, i, 0)),
            pl.BlockSpec((192, 128), lambda i: (0, 0)),
            pl.BlockSpec((1, 32), lambda i: (0, 0)),
            pl.BlockSpec((128, 256), lambda i: (0, 0)),
            pl.BlockSpec((1, 64), lambda i: (0, 0)),
            pl.BlockSpec((64, 576), lambda i: (0, 0)),
            pl.BlockSpec((1, 64), lambda i: (0, 0)),
        ],
        out_specs=pl.BlockSpec((BBC, 3136), lambda i: (i, 0)),
        out_shape=jax.ShapeDtypeStruct((bsz, 3136), bf16),
    )(x1, w1a, b1, w2a, b2, w3a, b3)

    hidden, logits, expert = pl.pallas_call(
        _dense_kernel,
        grid=(bsz // BB3,),
        in_specs=[
            pl.BlockSpec((BB3, 3136), lambda i: (i, 0)),
            pl.BlockSpec((3136, HIDDEN), lambda i: (0, 0)),
            pl.BlockSpec((1, HIDDEN), lambda i: (0, 0)),
            pl.BlockSpec((HIDDEN, NCAT), lambda i: (0, 0)),
            pl.BlockSpec((1, NCAT), lambda i: (0, 0)),
            pl.BlockSpec((HIDDEN, 2048), lambda i: (0, 0)),
            pl.BlockSpec((1, 2048), lambda i: (0, 0)),
            pl.BlockSpec((2048, 128), lambda i: (0, 0)),
            pl.BlockSpec((1, 128), lambda i: (0, 0)),
            pl.BlockSpec((128, NEXP), lambda i: (0, 0)),
        ],
        out_specs=[
            pl.BlockSpec((BB3, HIDDEN), lambda i: (i, 0)),
            pl.BlockSpec((BB3, NCAT), lambda i: (i, 0)),
            pl.BlockSpec((BB3, NEXP), lambda i: (i, 0)),
        ],
        out_shape=[
            jax.ShapeDtypeStruct((bsz, HIDDEN), f32),
            jax.ShapeDtypeStruct((bsz, NCAT), f32),
            jax.ShapeDtypeStruct((bsz, NEXP), f32),
        ],
    )(feats, pw, pb, cat_w, cb, wh1, bh1, w2bd, bh2, onehot)

    return (logits, expert, hidden)


# speedup vs baseline: 1.1580x; 1.0448x over previous
import functools

import jax
import jax.numpy as jnp
from jax import lax
from jax.experimental import pallas as pl
from jax.experimental.pallas import tpu as pltpu
from jax.experimental.pallas import tpu_sc as plsc

B = 512
NCAT = 16
NEXP = 64
EPC = 8
HIDDEN = 256

BBC = 8
BB3 = 256

NW = 32
NU = 21 * (B // 4)
UPW = NU // NW


def _sc_s2d_kernel(obs_hbm, x1_hbm, in0, in1, out0, out1, sem_g, sem_s):
    i32 = jnp.int32
    wid = lax.axis_index("s") * 2 + lax.axis_index("c")
    s16 = lax.iota(i32, 16)
    patt = 768 * (s16 // 4) + (s16 % 4)
    tail = s16 < 4

    ins = (in0, in1)
    outs = (out0, out1)

    def fire_gathers(u, slot):
        uid = wid * UPW + u
        p = uid // (B // 4)
        bg = uid - p * (B // 4)
        for r in range(48):
            b2, cc = r // 12, r % 12
            row = (bg * 4 + b2) * 252 + (cc * 21) + p
            pltpu.async_copy(obs_hbm.at[row], ins[slot].at[r], sem_g)

    def drain_gathers(slot):
        pltpu.make_async_copy(
            obs_hbm.at[pl.ds(0, 48)], ins[slot], sem_g).wait()

    def fire_scatters(u, slot):
        uid = wid * UPW + u
        p = uid // (B // 4)
        bg = uid - p * (B // 4)
        for q in range(21):
            dst = ((p * 21 + q) * B + bg * 4) * 192
            pltpu.async_copy(outs[slot].at[pl.ds(q * 768, 768)],
                             x1_hbm.at[pl.ds(dst, 768)], sem_s)

    def drain_scatters(slot):
        pltpu.make_async_copy(outs[slot],
                              x1_hbm.at[pl.ds(0, 16128)], sem_s).wait()

    def permute(slot):
        pv = [patt + 3072 * t for t in range(6)]
        tailcol = jnp.minimum(332 + s16, 335)

        def row_body(r, c2):
            b2s = r // 12
            ccs = r - b2s * 12
            ob0 = b2s * 192 + ccs * 16
            for i in range(4):
                ob = ob0 + i * 4
                for t in range(6):
                    if i == 3 and t == 5:
                        rv = lax.full((16,), 0, jnp.int32) + r
                        vals = plsc.load_gather(ins[slot], [rv, tailcol])
                    else:
                        vals = ins[slot][r, pl.ds(i * 84 + 16 * t, 16)]
                    oidx = pv[t] + ob
                    if t < 5:
                        plsc.store_scatter(outs[slot], [oidx], vals)
                    else:
                        plsc.store_scatter(outs[slot], [oidx], vals,
                                           mask=tail)
            return c2

        lax.fori_loop(0, 48, row_body, 0)

    fire_gathers(wid * 0, 0)

    def pair_body(g, carry):
        for ph in range(2):
            u = g * 2 + ph
            drain_gathers(ph)

            @pl.when(u + 1 < UPW)
            def _():
                fire_gathers(u + 1, 1 - ph)

            @pl.when(u >= 2)
            def _():
                drain_scatters(ph)

            permute(ph)
            fire_scatters(u, ph)
        return carry

    lax.fori_loop(0, UPW // 2, pair_body, 0)
    drain_scatters(0)
    drain_scatters(1)


def _sc_s2d(obs):
    obs_rows = obs.reshape(B * 12 * 21, 336)
    mesh = plsc.VectorSubcoreMesh(core_axis_name="c", subcore_axis_name="s")
    k = functools.partial(
        pl.kernel, _sc_s2d_kernel, mesh=mesh,
        compiler_params=pltpu.CompilerParams(needs_layout_passes=False),
        out_type=jax.ShapeDtypeStruct((21 * 21 * B * 192,), jnp.float32),
        scratch_types=[
            pltpu.VMEM((48, 336), jnp.float32),
            pltpu.VMEM((48, 336), jnp.float32),
            pltpu.VMEM((16128,), jnp.float32),
            pltpu.VMEM((16128,), jnp.float32),
            pltpu.SemaphoreType.DMA,
            pltpu.SemaphoreType.DMA,
        ],
    )()
    return k(obs_rows).reshape(21, 21, B, 192)


def _convs_kernel(x_ref, w1_ref, b1_ref, w2_ref, b2_ref, w3_ref, b3_ref,
                  f_ref):
    bb = x_ref.shape[2]
    bf16 = jnp.bfloat16
    x = x_ref[...].astype(bf16).reshape(21 * 21 * bb, 192)
    y1 = jnp.dot(x, w1_ref[...], preferred_element_type=jnp.float32)
    y1 = y1.reshape(21, 21, bb, 128)
    o1 = (y1[0:20, 0:20, :, 0:32] + y1[0:20, 1:21, :, 32:64]
          + y1[1:21, 0:20, :, 64:96] + y1[1:21, 1:21, :, 96:128])
    o1 = jnp.maximum(o1 + b1_ref[...].reshape(1, 1, 1, 32), 0.0)
    o1r = o1.astype(bf16).reshape(10, 2, 10, 2, bb, 32)
    x2 = jnp.concatenate(
        [o1r[:, i, :, j] for i in range(2) for j in range(2)], axis=-1)
    y2 = jnp.dot(x2.reshape(100 * bb, 128), w2_ref[...],
                 preferred_element_type=jnp.float32)
    y2 = y2.reshape(10, 10, bb, 256)
    o2 = (y2[0:9, 0:9, :, 0:64] + y2[0:9, 1:10, :, 64:128]
          + y2[1:10, 0:9, :, 128:192] + y2[1:10, 1:10, :, 192:256])
    o2 = jnp.maximum(o2 + b2_ref[...].reshape(1, 1, 1, 64), 0.0)
    y3 = jnp.dot(o2.astype(bf16).reshape(81 * bb, 64), w3_ref[...],
                 preferred_element_type=jnp.float32)
    y3 = y3.reshape(9, 9, bb, 576)
    o3 = 0.0
    for kh in range(3):
        for kw in range(3):
            g = (kh * 3 + kw) * 64
            o3 = o3 + y3[kh:kh + 7, kw:kw + 7, :, g:g + 64]
    o3 = jnp.maximum(o3 + b3_ref[...].reshape(1, 1, 1, 64), 0.0)
    f_ref[...] = o3.astype(bf16).transpose(2, 0, 1, 3).reshape(bb, 3136)


def _dense_kernel(f_ref, pw_ref, pb_ref, cw_ref, cb_ref, w1_ref, b1_ref,
                  w2_ref, b2_ref, oh_ref, hid_ref, log_ref, exp_ref):
    f = f_ref[...]
    hid = jnp.maximum(
        jnp.dot(f, pw_ref[...], preferred_element_type=jnp.float32)
        + pb_ref[...], 0.0)
    hid_ref[...] = hid
    logits = jnp.dot(hid, cw_ref[...], preferred_element_type=jnp.float32) \
        + cb_ref[...]
    log_ref[...] = logits
    m = jnp.max(logits, axis=-1, keepdims=True)
    e = jnp.exp(logits - m)
    probs = e / jnp.sum(e, axis=-1, keepdims=True)

    h1 = jnp.maximum(
        jnp.dot(hid, w1_ref[...], preferred_element_type=jnp.float32)
        + b1_ref[...], 0.0)
    z = jnp.dot(h1, w2_ref[...], preferred_element_type=jnp.float32) \
        + b2_ref[...]
    local = jax.nn.sigmoid(z)
    bb = f.shape[0]
    wts = jnp.broadcast_to(probs[:, :, None], (bb, NCAT, EPC))
    weighted = wts.reshape(bb, NCAT * EPC) * local
    exp_ref[...] = jnp.dot(weighted, oh_ref[...],
                           preferred_element_type=jnp.float32)


def kernel(obs, conv1_w, conv1_b, conv2_w, conv2_b, conv3_w, conv3_b,
           proj_w, proj_b, cat_w, cat_b, head_w1, head_b1, head_w2,
           head_b2, mapping):
    f32 = jnp.float32
    bf16 = jnp.bfloat16
    bsz = obs.shape[0]

    w1a = conv1_w.reshape(32, 12, 2, 4, 2, 4).transpose(1, 3, 5, 2, 4, 0)
    w1a = w1a.reshape(192, 128).astype(bf16)
    b1 = conv1_b.reshape(1, 32)

    w2a = conv2_w.reshape(64, 32, 2, 2, 2, 2).transpose(3, 5, 1, 2, 4, 0)
    w2a = w2a.reshape(128, 256).astype(bf16)
    b2 = conv2_b.reshape(1, 64)

    w3a = conv3_w.transpose(1, 2, 3, 0).reshape(64, 576).astype(bf16)
    b3 = conv3_b.reshape(1, 64)

    pw = proj_w.reshape(64, 7, 7, HIDDEN).transpose(1, 2, 0, 3)
    pw = pw.reshape(7 * 7 * 64, HIDDEN).astype(bf16)
    pb = proj_b.reshape(1, HIDDEN)
    cb = cat_b.reshape(1, NCAT)

    wh1 = head_w1.transpose(1, 0, 2).reshape(HIDDEN, NCAT * (HIDDEN // 2))
    bh1 = head_b1.reshape(1, NCAT * (HIDDEN // 2))
    eye = jnp.eye(NCAT, dtype=f32)
    w2bd = (eye[:, None, :, None] * head_w2[:, :, None, :])
    w2bd = w2bd.reshape(NCAT * (HIDDEN // 2), NCAT * EPC)
    bh2 = head_b2.reshape(1, NCAT * EPC)

    onehot = (mapping.reshape(-1)[:, None]
              == jnp.arange(NEXP, dtype=jnp.int32)[None, :]).astype(f32)

    x1 = _sc_s2d(obs)

    feats = pl.pallas_call(
        _convs_kernel,
        grid=(bsz // BBC,),
        in_specs=[
            pl.BlockSpec((21, 21, BBC, 192), lambda i: (0, 0, i, 0)),
            pl.BlockSpec((192, 128), lambda i: (0, 0)),
            pl.BlockSpec((1, 32), lambda i: (0, 0)),
            pl.BlockSpec((128, 256), lambda i: (0, 0)),
            pl.BlockSpec((1, 64), lambda i: (0, 0)),
            pl.BlockSpec((64, 576), lambda i: (0, 0)),
            pl.BlockSpec((1, 64), lambda i: (0, 0)),
        ],
        out_specs=pl.BlockSpec((BBC, 3136), lambda i: (i, 0)),
        out_shape=jax.ShapeDtypeStruct((bsz, 3136), bf16),
    )(x1, w1a, b1, w2a, b2, w3a, b3)

    hidden, logits, expert = pl.pallas_call(
        _dense_kernel,
        grid=(bsz // BB3,),
        in_specs=[
            pl.BlockSpec((BB3, 3136), lambda i: (i, 0)),
            pl.BlockSpec((3136, HIDDEN), lambda i: (0, 0)),
            pl.BlockSpec((1, HIDDEN), lambda i: (0, 0)),
            pl.BlockSpec((HIDDEN, NCAT), lambda i: (0, 0)),
            pl.BlockSpec((1, NCAT), lambda i: (0, 0)),
            pl.BlockSpec((HIDDEN, 2048), lambda i: (0, 0)),
            pl.BlockSpec((1, 2048), lambda i: (0, 0)),
            pl.BlockSpec((2048, 128), lambda i: (0, 0)),
            pl.BlockSpec((1, 128), lambda i: (0, 0)),
            pl.BlockSpec((128, NEXP), lambda i: (0, 0)),
        ],
        out_specs=[
            pl.BlockSpec((BB3, HIDDEN), lambda i: (i, 0)),
            pl.BlockSpec((BB3, NCAT), lambda i: (i, 0)),
            pl.BlockSpec((BB3, NEXP), lambda i: (i, 0)),
        ],
        out_shape=[
            jax.ShapeDtypeStruct((bsz, HIDDEN), f32),
            jax.ShapeDtypeStruct((bsz, NCAT), f32),
            jax.ShapeDtypeStruct((bsz, NEXP), f32),
        ],
    )(feats, pw, pb, cat_w, cb, wh1, bh1, w2bd, bh2, onehot)

    return (logits, expert, hidden)


# speedup vs baseline: 1.7143x; 1.4803x over previous
import jax
import jax.numpy as jnp
from jax.experimental import pallas as pl

B = 512
NCAT = 16
NEXP = 64
EPC = 8
HIDDEN = 256

BBC = 8
BB3 = 256


def _convs_kernel(x_ref, w1_ref, b1_ref, w2_ref, b2_ref, w3_ref, b3_ref,
                  f_ref):
    bb = x_ref.shape[2]
    bf16 = jnp.bfloat16
    x = x_ref[...].reshape(21 * 21 * bb, 192)
    y1 = jnp.dot(x, w1_ref[...], preferred_element_type=jnp.float32)
    y1 = y1.reshape(21, 21, bb, 128)
    o1 = (y1[0:20, 0:20, :, 0:32] + y1[0:20, 1:21, :, 32:64]
          + y1[1:21, 0:20, :, 64:96] + y1[1:21, 1:21, :, 96:128])
    o1 = jnp.maximum(o1 + b1_ref[...].reshape(1, 1, 1, 32), 0.0)
    o1r = o1.astype(bf16).reshape(10, 2, 10, 2, bb, 32)
    x2 = jnp.concatenate(
        [o1r[:, i, :, j] for i in range(2) for j in range(2)], axis=-1)
    y2 = jnp.dot(x2.reshape(100 * bb, 128), w2_ref[...],
                 preferred_element_type=jnp.float32)
    y2 = y2.reshape(10, 10, bb, 256)
    o2 = (y2[0:9, 0:9, :, 0:64] + y2[0:9, 1:10, :, 64:128]
          + y2[1:10, 0:9, :, 128:192] + y2[1:10, 1:10, :, 192:256])
    o2 = jnp.maximum(o2 + b2_ref[...].reshape(1, 1, 1, 64), 0.0)
    y3 = jnp.dot(o2.astype(bf16).reshape(81 * bb, 64), w3_ref[...],
                 preferred_element_type=jnp.float32)
    y3 = y3.reshape(9, 9, bb, 576)
    o3 = 0.0
    for kh in range(3):
        for kw in range(3):
            g = (kh * 3 + kw) * 64
            o3 = o3 + y3[kh:kh + 7, kw:kw + 7, :, g:g + 64]
    o3 = jnp.maximum(o3 + b3_ref[...].reshape(1, 1, 1, 64), 0.0)
    f_ref[...] = o3.astype(bf16).transpose(2, 0, 1, 3).reshape(bb, 3136)


def _dense_kernel(f_ref, pw_ref, pb_ref, cw_ref, cb_ref, w1_ref, b1_ref,
                  w2_ref, b2_ref, oh_ref, hid_ref, log_ref, exp_ref):
    f = f_ref[...]
    hid = jnp.maximum(
        jnp.dot(f, pw_ref[...], preferred_element_type=jnp.float32)
        + pb_ref[...], 0.0)
    hid_ref[...] = hid
    logits = jnp.dot(hid, cw_ref[...], preferred_element_type=jnp.float32) \
        + cb_ref[...]
    log_ref[...] = logits
    m = jnp.max(logits, axis=-1, keepdims=True)
    e = jnp.exp(logits - m)
    probs = e / jnp.sum(e, axis=-1, keepdims=True)

    h1 = jnp.maximum(
        jnp.dot(hid, w1_ref[...], preferred_element_type=jnp.float32)
        + b1_ref[...], 0.0)
    z = jnp.dot(h1, w2_ref[...], preferred_element_type=jnp.float32) \
        + b2_ref[...]
    local = jax.nn.sigmoid(z)
    bb = f.shape[0]
    wts = jnp.broadcast_to(probs[:, :, None], (bb, NCAT, EPC))
    weighted = wts.reshape(bb, NCAT * EPC) * local
    exp_ref[...] = jnp.dot(weighted, oh_ref[...],
                           preferred_element_type=jnp.float32)


def kernel(obs, conv1_w, conv1_b, conv2_w, conv2_b, conv3_w, conv3_b,
           proj_w, proj_b, cat_w, cat_b, head_w1, head_b1, head_w2,
           head_b2, mapping):
    f32 = jnp.float32
    bf16 = jnp.bfloat16
    bsz = obs.shape[0]

    w1a = conv1_w.reshape(32, 12, 2, 4, 2, 4).transpose(1, 3, 5, 2, 4, 0)
    w1a = w1a.reshape(192, 128).astype(bf16)
    b1 = conv1_b.reshape(1, 32)

    w2a = conv2_w.reshape(64, 32, 2, 2, 2, 2).transpose(3, 5, 1, 2, 4, 0)
    w2a = w2a.reshape(128, 256).astype(bf16)
    b2 = conv2_b.reshape(1, 64)

    w3a = conv3_w.transpose(1, 2, 3, 0).reshape(64, 576).astype(bf16)
    b3 = conv3_b.reshape(1, 64)

    pw = proj_w.reshape(64, 7, 7, HIDDEN).transpose(1, 2, 0, 3)
    pw = pw.reshape(7 * 7 * 64, HIDDEN).astype(bf16)
    pb = proj_b.reshape(1, HIDDEN)
    cb = cat_b.reshape(1, NCAT)

    wh1 = head_w1.transpose(1, 0, 2).reshape(HIDDEN, NCAT * (HIDDEN // 2))
    bh1 = head_b1.reshape(1, NCAT * (HIDDEN // 2))
    eye = jnp.eye(NCAT, dtype=f32)
    w2bd = (eye[:, None, :, None] * head_w2[:, :, None, :])
    w2bd = w2bd.reshape(NCAT * (HIDDEN // 2), NCAT * EPC)
    bh2 = head_b2.reshape(1, NCAT * EPC)

    onehot = (mapping.reshape(-1)[:, None]
              == jnp.arange(NEXP, dtype=jnp.int32)[None, :]).astype(f32)

    half = bsz // 2
    fparts = []
    for h in range(2):
        oh_ = obs[h * half:(h + 1) * half]
        x1 = oh_.reshape(half, 12, 21, 4, 21, 4).transpose(2, 4, 0, 1, 3, 5)
        x1 = x1.reshape(21, 21, half, 192).astype(bf16)
        fparts.append(pl.pallas_call(
            _convs_kernel,
            grid=(half // BBC,),
            in_specs=[
                pl.BlockSpec((21, 21, BBC, 192), lambda i: (0, 0, i, 0)),
                pl.BlockSpec((192, 128), lambda i: (0, 0)),
                pl.BlockSpec((1, 32), lambda i: (0, 0)),
                pl.BlockSpec((128, 256), lambda i: (0, 0)),
                pl.BlockSpec((1, 64), lambda i: (0, 0)),
                pl.BlockSpec((64, 576), lambda i: (0, 0)),
                pl.BlockSpec((1, 64), lambda i: (0, 0)),
            ],
            out_specs=pl.BlockSpec((BBC, 3136), lambda i: (i, 0)),
            out_shape=jax.ShapeDtypeStruct((half, 3136), bf16),
        )(x1, w1a, b1, w2a, b2, w3a, b3))
    feats = jnp.concatenate(fparts, axis=0)

    hidden, logits, expert = pl.pallas_call(
        _dense_kernel,
        grid=(bsz // BB3,),
        in_specs=[
            pl.BlockSpec((BB3, 3136), lambda i: (i, 0)),
            pl.BlockSpec((3136, HIDDEN), lambda i: (0, 0)),
            pl.BlockSpec((1, HIDDEN), lambda i: (0, 0)),
            pl.BlockSpec((HIDDEN, NCAT), lambda i: (0, 0)),
            pl.BlockSpec((1, NCAT), lambda i: (0, 0)),
            pl.BlockSpec((HIDDEN, 2048), lambda i: (0, 0)),
            pl.BlockSpec((1, 2048), lambda i: (0, 0)),
            pl.BlockSpec((2048, 128), lambda i: (0, 0)),
            pl.BlockSpec((1, 128), lambda i: (0, 0)),
            pl.BlockSpec((128, NEXP), lambda i: (0, 0)),
        ],
        out_specs=[
            pl.BlockSpec((BB3, HIDDEN), lambda i: (i, 0)),
            pl.BlockSpec((BB3, NCAT), lambda i: (i, 0)),
            pl.BlockSpec((BB3, NEXP), lambda i: (i, 0)),
        ],
        out_shape=[
            jax.ShapeDtypeStruct((bsz, HIDDEN), f32),
            jax.ShapeDtypeStruct((bsz, NCAT), f32),
            jax.ShapeDtypeStruct((bsz, NEXP), f32),
        ],
    )(feats, pw, pb, cat_w, cb, wh1, bh1, w2bd, bh2, onehot)

    return (logits, expert, hidden)


# speedup vs baseline: 1.8410x; 1.0739x over previous
import jax
import jax.numpy as jnp
from jax.experimental import pallas as pl

B = 512
NCAT = 16
NEXP = 64
EPC = 8
HIDDEN = 256

BBC = 16
BB3 = 256


def _convs_kernel(x_ref, w1_ref, b1_ref, w2_ref, b2_ref, w3_ref, b3_ref,
                  f_ref):
    bb = x_ref.shape[2]
    bf16 = jnp.bfloat16
    x = x_ref[...].reshape(21 * 21 * bb, 192)
    y1 = jnp.dot(x, w1_ref[...], preferred_element_type=jnp.float32)
    y1 = y1.reshape(21, 21, bb, 128)
    o1 = (y1[0:20, 0:20, :, 0:32] + y1[0:20, 1:21, :, 32:64]
          + y1[1:21, 0:20, :, 64:96] + y1[1:21, 1:21, :, 96:128])
    o1 = jnp.maximum(o1 + b1_ref[...].reshape(1, 1, 1, 32), 0.0)
    o1r = o1.astype(bf16).reshape(10, 2, 10, 2, bb, 32)
    x2 = jnp.concatenate(
        [o1r[:, i, :, j] for i in range(2) for j in range(2)], axis=-1)
    y2 = jnp.dot(x2.reshape(100 * bb, 128), w2_ref[...],
                 preferred_element_type=jnp.float32)
    y2 = y2.reshape(10, 10, bb, 256)
    o2 = (y2[0:9, 0:9, :, 0:64] + y2[0:9, 1:10, :, 64:128]
          + y2[1:10, 0:9, :, 128:192] + y2[1:10, 1:10, :, 192:256])
    o2 = jnp.maximum(o2 + b2_ref[...].reshape(1, 1, 1, 64), 0.0)
    y3 = jnp.dot(o2.astype(bf16).reshape(81 * bb, 64), w3_ref[...],
                 preferred_element_type=jnp.float32)
    y3 = y3.reshape(9, 9, bb, 576)
    o3 = 0.0
    for kh in range(3):
        for kw in range(3):
            g = (kh * 3 + kw) * 64
            o3 = o3 + y3[kh:kh + 7, kw:kw + 7, :, g:g + 64]
    o3 = jnp.maximum(o3 + b3_ref[...].reshape(1, 1, 1, 64), 0.0)
    f_ref[...] = o3.astype(bf16).transpose(2, 0, 1, 3).reshape(bb, 3136)


def _dense_kernel(f_ref, pw_ref, pb_ref, cw_ref, cb_ref, w1_ref, b1_ref,
                  w2_ref, b2_ref, oh_ref, hid_ref, log_ref, exp_ref):
    f = f_ref[...]
    hid = jnp.maximum(
        jnp.dot(f, pw_ref[...], preferred_element_type=jnp.float32)
        + pb_ref[...], 0.0)
    hid_ref[...] = hid
    logits = jnp.dot(hid, cw_ref[...], preferred_element_type=jnp.float32) \
        + cb_ref[...]
    log_ref[...] = logits
    m = jnp.max(logits, axis=-1, keepdims=True)
    e = jnp.exp(logits - m)
    probs = e / jnp.sum(e, axis=-1, keepdims=True)

    h1 = jnp.maximum(
        jnp.dot(hid, w1_ref[...], preferred_element_type=jnp.float32)
        + b1_ref[...], 0.0)
    z = jnp.dot(h1, w2_ref[...], preferred_element_type=jnp.float32) \
        + b2_ref[...]
    local = jax.nn.sigmoid(z)
    bb = f.shape[0]
    wts = jnp.broadcast_to(probs[:, :, None], (bb, NCAT, EPC))
    weighted = wts.reshape(bb, NCAT * EPC) * local
    exp_ref[...] = jnp.dot(weighted, oh_ref[...],
                           preferred_element_type=jnp.float32)


def kernel(obs, conv1_w, conv1_b, conv2_w, conv2_b, conv3_w, conv3_b,
           proj_w, proj_b, cat_w, cat_b, head_w1, head_b1, head_w2,
           head_b2, mapping):
    f32 = jnp.float32
    bf16 = jnp.bfloat16
    bsz = obs.shape[0]

    x1 = obs.reshape(bsz, 12, 21, 4, 21, 4).transpose(2, 4, 0, 1, 3, 5)
    x1 = x1.reshape(21, 21, bsz, 192).astype(bf16)
    w1a = conv1_w.reshape(32, 12, 2, 4, 2, 4).transpose(1, 3, 5, 2, 4, 0)
    w1a = w1a.reshape(192, 128).astype(bf16)
    b1 = conv1_b.reshape(1, 32)

    w2a = conv2_w.reshape(64, 32, 2, 2, 2, 2).transpose(3, 5, 1, 2, 4, 0)
    w2a = w2a.reshape(128, 256).astype(bf16)
    b2 = conv2_b.reshape(1, 64)

    w3a = conv3_w.transpose(1, 2, 3, 0).reshape(64, 576).astype(bf16)
    b3 = conv3_b.reshape(1, 64)

    pw = proj_w.reshape(64, 7, 7, HIDDEN).transpose(1, 2, 0, 3)
    pw = pw.reshape(7 * 7 * 64, HIDDEN).astype(bf16)
    pb = proj_b.reshape(1, HIDDEN)
    cb = cat_b.reshape(1, NCAT)

    wh1 = head_w1.transpose(1, 0, 2).reshape(HIDDEN, NCAT * (HIDDEN // 2))
    bh1 = head_b1.reshape(1, NCAT * (HIDDEN // 2))
    eye = jnp.eye(NCAT, dtype=f32)
    w2bd = (eye[:, None, :, None] * head_w2[:, :, None, :])
    w2bd = w2bd.reshape(NCAT * (HIDDEN // 2), NCAT * EPC)
    bh2 = head_b2.reshape(1, NCAT * EPC)

    onehot = (mapping.reshape(-1)[:, None]
              == jnp.arange(NEXP, dtype=jnp.int32)[None, :]).astype(f32)

    feats = pl.pallas_call(
        _convs_kernel,
        grid=(bsz // BBC,),
        in_specs=[
            pl.BlockSpec((21, 21, BBC, 192), lambda i: (0, 0, i, 0)),
            pl.BlockSpec((192, 128), lambda i: (0, 0)),
            pl.BlockSpec((1, 32), lambda i: (0, 0)),
            pl.BlockSpec((128, 256), lambda i: (0, 0)),
            pl.BlockSpec((1, 64), lambda i: (0, 0)),
            pl.BlockSpec((64, 576), lambda i: (0, 0)),
            pl.BlockSpec((1, 64), lambda i: (0, 0)),
        ],
        out_specs=pl.BlockSpec((BBC, 3136), lambda i: (i, 0)),
        out_shape=jax.ShapeDtypeStruct((bsz, 3136), bf16),
    )(x1, w1a, b1, w2a, b2, w3a, b3)

    hidden, logits, expert = pl.pallas_call(
        _dense_kernel,
        grid=(bsz // BB3,),
        in_specs=[
            pl.BlockSpec((BB3, 3136), lambda i: (i, 0)),
            pl.BlockSpec((3136, HIDDEN), lambda i: (0, 0)),
            pl.BlockSpec((1, HIDDEN), lambda i: (0, 0)),
            pl.BlockSpec((HIDDEN, NCAT), lambda i: (0, 0)),
            pl.BlockSpec((1, NCAT), lambda i: (0, 0)),
            pl.BlockSpec((HIDDEN, 2048), lambda i: (0, 0)),
            pl.BlockSpec((1, 2048), lambda i: (0, 0)),
            pl.BlockSpec((2048, 128), lambda i: (0, 0)),
            pl.BlockSpec((1, 128), lambda i: (0, 0)),
            pl.BlockSpec((128, NEXP), lambda i: (0, 0)),
        ],
        out_specs=[
            pl.BlockSpec((BB3, HIDDEN), lambda i: (i, 0)),
            pl.BlockSpec((BB3, NCAT), lambda i: (i, 0)),
            pl.BlockSpec((BB3, NEXP), lambda i: (i, 0)),
        ],
        out_shape=[
            jax.ShapeDtypeStruct((bsz, HIDDEN), f32),
            jax.ShapeDtypeStruct((bsz, NCAT), f32),
            jax.ShapeDtypeStruct((bsz, NEXP), f32),
        ],
    )(feats, pw, pb, cat_w, cb, wh1, bh1, w2bd, bh2, onehot)

    return (logits, expert, hidden)


# speedup vs baseline: 1.8645x; 1.0128x over previous
import jax
import jax.numpy as jnp
from jax.experimental import pallas as pl

B = 512
NCAT = 16
NEXP = 64
EPC = 8
HIDDEN = 256

BBC = 32
BB3 = 256


def _convs_kernel(x_ref, w1_ref, b1_ref, w2_ref, b2_ref, w3_ref, b3_ref,
                  f_ref):
    bb = x_ref.shape[2]
    bf16 = jnp.bfloat16
    x = x_ref[...].reshape(21 * 21 * bb, 192)
    y1 = jnp.dot(x, w1_ref[...], preferred_element_type=jnp.float32)
    y1 = y1.reshape(21, 21, bb, 128)
    o1 = (y1[0:20, 0:20, :, 0:32] + y1[0:20, 1:21, :, 32:64]
          + y1[1:21, 0:20, :, 64:96] + y1[1:21, 1:21, :, 96:128])
    o1 = jnp.maximum(o1 + b1_ref[...].reshape(1, 1, 1, 32), 0.0)
    o1r = o1.astype(bf16).reshape(10, 2, 10, 2, bb, 32)
    x2 = jnp.concatenate(
        [o1r[:, i, :, j] for i in range(2) for j in range(2)], axis=-1)
    y2 = jnp.dot(x2.reshape(100 * bb, 128), w2_ref[...],
                 preferred_element_type=jnp.float32)
    y2 = y2.reshape(10, 10, bb, 256)
    o2 = (y2[0:9, 0:9, :, 0:64] + y2[0:9, 1:10, :, 64:128]
          + y2[1:10, 0:9, :, 128:192] + y2[1:10, 1:10, :, 192:256])
    o2 = jnp.maximum(o2 + b2_ref[...].reshape(1, 1, 1, 64), 0.0)
    y3 = jnp.dot(o2.astype(bf16).reshape(81 * bb, 64), w3_ref[...],
                 preferred_element_type=jnp.float32)
    y3 = y3.reshape(9, 9, bb, 576)
    o3 = 0.0
    for kh in range(3):
        for kw in range(3):
            g = (kh * 3 + kw) * 64
            o3 = o3 + y3[kh:kh + 7, kw:kw + 7, :, g:g + 64]
    o3 = jnp.maximum(o3 + b3_ref[...].reshape(1, 1, 1, 64), 0.0)
    f_ref[...] = o3.astype(bf16).transpose(2, 0, 1, 3).reshape(bb, 3136)


def _dense_kernel(f_ref, pw_ref, pb_ref, cw_ref, cb_ref, w1_ref, b1_ref,
                  w2_ref, b2_ref, oh_ref, hid_ref, log_ref, exp_ref):
    f = f_ref[...]
    hid = jnp.maximum(
        jnp.dot(f, pw_ref[...], preferred_element_type=jnp.float32)
        + pb_ref[...], 0.0)
    hid_ref[...] = hid
    logits = jnp.dot(hid, cw_ref[...], preferred_element_type=jnp.float32) \
        + cb_ref[...]
    log_ref[...] = logits
    m = jnp.max(logits, axis=-1, keepdims=True)
    e = jnp.exp(logits - m)
    probs = e / jnp.sum(e, axis=-1, keepdims=True)

    h1 = jnp.maximum(
        jnp.dot(hid, w1_ref[...], preferred_element_type=jnp.float32)
        + b1_ref[...], 0.0)
    z = jnp.dot(h1, w2_ref[...], preferred_element_type=jnp.float32) \
        + b2_ref[...]
    local = jax.nn.sigmoid(z)
    bb = f.shape[0]
    wts = jnp.broadcast_to(probs[:, :, None], (bb, NCAT, EPC))
    weighted = wts.reshape(bb, NCAT * EPC) * local
    exp_ref[...] = jnp.dot(weighted, oh_ref[...],
                           preferred_element_type=jnp.float32)


def kernel(obs, conv1_w, conv1_b, conv2_w, conv2_b, conv3_w, conv3_b,
           proj_w, proj_b, cat_w, cat_b, head_w1, head_b1, head_w2,
           head_b2, mapping):
    f32 = jnp.float32
    bf16 = jnp.bfloat16
    bsz = obs.shape[0]

    x1 = obs.reshape(bsz, 12, 21, 4, 21, 4).transpose(2, 4, 0, 1, 3, 5)
    x1 = x1.reshape(21, 21, bsz, 192).astype(bf16)
    w1a = conv1_w.reshape(32, 12, 2, 4, 2, 4).transpose(1, 3, 5, 2, 4, 0)
    w1a = w1a.reshape(192, 128).astype(bf16)
    b1 = conv1_b.reshape(1, 32)

    w2a = conv2_w.reshape(64, 32, 2, 2, 2, 2).transpose(3, 5, 1, 2, 4, 0)
    w2a = w2a.reshape(128, 256).astype(bf16)
    b2 = conv2_b.reshape(1, 64)

    w3a = conv3_w.transpose(1, 2, 3, 0).reshape(64, 576).astype(bf16)
    b3 = conv3_b.reshape(1, 64)

    pw = proj_w.reshape(64, 7, 7, HIDDEN).transpose(1, 2, 0, 3)
    pw = pw.reshape(7 * 7 * 64, HIDDEN).astype(bf16)
    pb = proj_b.reshape(1, HIDDEN)
    cb = cat_b.reshape(1, NCAT)

    wh1 = head_w1.transpose(1, 0, 2).reshape(HIDDEN, NCAT * (HIDDEN // 2))
    bh1 = head_b1.reshape(1, NCAT * (HIDDEN // 2))
    eye = jnp.eye(NCAT, dtype=f32)
    w2bd = (eye[:, None, :, None] * head_w2[:, :, None, :])
    w2bd = w2bd.reshape(NCAT * (HIDDEN // 2), NCAT * EPC)
    bh2 = head_b2.reshape(1, NCAT * EPC)

    onehot = (mapping.reshape(-1)[:, None]
              == jnp.arange(NEXP, dtype=jnp.int32)[None, :]).astype(f32)

    feats = pl.pallas_call(
        _convs_kernel,
        grid=(bsz // BBC,),
        in_specs=[
            pl.BlockSpec((21, 21, BBC, 192), lambda i: (0, 0, i, 0)),
            pl.BlockSpec((192, 128), lambda i: (0, 0)),
            pl.BlockSpec((1, 32), lambda i: (0, 0)),
            pl.BlockSpec((128, 256), lambda i: (0, 0)),
            pl.BlockSpec((1, 64), lambda i: (0, 0)),
            pl.BlockSpec((64, 576), lambda i: (0, 0)),
            pl.BlockSpec((1, 64), lambda i: (0, 0)),
        ],
        out_specs=pl.BlockSpec((BBC, 3136), lambda i: (i, 0)),
        out_shape=jax.ShapeDtypeStruct((bsz, 3136), bf16),
    )(x1, w1a, b1, w2a, b2, w3a, b3)

    hidden, logits, expert = pl.pallas_call(
        _dense_kernel,
        grid=(bsz // BB3,),
        in_specs=[
            pl.BlockSpec((BB3, 3136), lambda i: (i, 0)),
            pl.BlockSpec((3136, HIDDEN), lambda i: (0, 0)),
            pl.BlockSpec((1, HIDDEN), lambda i: (0, 0)),
            pl.BlockSpec((HIDDEN, NCAT), lambda i: (0, 0)),
            pl.BlockSpec((1, NCAT), lambda i: (0, 0)),
            pl.BlockSpec((HIDDEN, 2048), lambda i: (0, 0)),
            pl.BlockSpec((1, 2048), lambda i: (0, 0)),
            pl.BlockSpec((2048, 128), lambda i: (0, 0)),
            pl.BlockSpec((1, 128), lambda i: (0, 0)),
            pl.BlockSpec((128, NEXP), lambda i: (0, 0)),
        ],
        out_specs=[
            pl.BlockSpec((BB3, HIDDEN), lambda i: (i, 0)),
            pl.BlockSpec((BB3, NCAT), lambda i: (i, 0)),
            pl.BlockSpec((BB3, NEXP), lambda i: (i, 0)),
        ],
        out_shape=[
            jax.ShapeDtypeStruct((bsz, HIDDEN), f32),
            jax.ShapeDtypeStruct((bsz, NCAT), f32),
            jax.ShapeDtypeStruct((bsz, NEXP), f32),
        ],
    )(feats, pw, pb, cat_w, cb, wh1, bh1, w2bd, bh2, onehot)

    return (logits, expert, hidden)
